# baseline (speedup 1.0000x reference)
import functools

import jax
import jax.numpy as jnp
import numpy as np
from jax import lax
from jax.experimental import pallas as pl
from jax.experimental.pallas import tpu as pltpu
from jax.experimental.pallas import tpu_sc as plsc

_BS = 2
_Q = 10000
_D = 256
_NH = 8
_NP = 4
_HD = 32
_SIDE = 100
_ROWS = _BS * _Q
_T = 400
_GRID = _ROWS // _T
_NW = 32
_CH = 8
_NCHUNK = _ROWS // _CH


_TPAD = 104


def _vq_body(v_ref, w_ref, b_ref, o_ref):
    r0 = pl.program_id(0) * _T
    m = (jnp.dot(v_ref[pl.ds(r0, _T + _TPAD), :], w_ref[...],
                 preferred_element_type=jnp.float32) + b_ref[...])
    shifts = (m[0:_T], m[1:_T + 1], m[100:_T + 100], m[101:_T + 101])
    parts = []
    for h in range(_NH):
        for s in shifts:
            parts.append(s[:, h * _HD:(h + 1) * _HD])
    o_ref[...] = jnp.concatenate(parts, axis=1)


def _vq(v_pad, w, b):
    return pl.pallas_call(
        _vq_body,
        grid=(_GRID,),
        in_specs=[
            pl.BlockSpec((_ROWS + _TPAD, _D), lambda i: (0, 0)),
            pl.BlockSpec((_D, _D), lambda i: (0, 0)),
            pl.BlockSpec((1, _D), lambda i: (0, 0)),
        ],
        out_specs=pl.BlockSpec((_T, 4 * _D), lambda i: (i, 0)),
        out_shape=jax.ShapeDtypeStruct((_ROWS, 4 * _D), jnp.float32),
    )(v_pad, w, b)


def _mm_bias_res_body(x_ref, w_ref, b_ref, id_ref, o_ref):
    o_ref[...] = (jnp.dot(x_ref[...], w_ref[...],
                          preferred_element_type=jnp.float32)
                  + b_ref[...] + id_ref[...])


def _mm_bias_res(x, w, b, ident):
    return pl.pallas_call(
        _mm_bias_res_body,
        grid=(_GRID,),
        in_specs=[
            pl.BlockSpec((_T, _D), lambda i: (i, 0)),
            pl.BlockSpec((_D, _D), lambda i: (0, 0)),
            pl.BlockSpec((1, _D), lambda i: (0, 0)),
            pl.BlockSpec((_T, _D), lambda i: (i, 0)),
        ],
        out_specs=pl.BlockSpec((_T, _D), lambda i: (i, 0)),
        out_shape=jax.ShapeDtypeStruct((_ROWS, _D), jnp.float32),
    )(x, w, b, ident)


def _prep_body(q_ref, r_ref, wcat_ref, bcat_ref, s_ref, idx_ref, cw_ref):
    t = pl.program_id(0)
    b = t // (_GRID // _BS)
    s = (jnp.dot(q_ref[...], wcat_ref[...],
                 preferred_element_type=jnp.float32) + bcat_ref[...])
    ox = s[:, 0:32]
    oy = s[:, 32:64]
    lg = s[:, 64:96]
    e = jnp.exp(lg)
    denom = jnp.dot(e, s_ref[...], preferred_element_type=jnp.float32)
    attn = e / denom
    rx = r_ref[:, 0:1]
    ry = r_ref[:, 1:2]
    gx = rx * float(_SIDE) - 0.5 + ox
    gy = ry * float(_SIDE) - 0.5 + oy
    x0 = jnp.floor(gx)
    fx = gx - x0
    y0 = jnp.floor(gy)
    fy = gy - y0
    lane = lax.broadcasted_iota(jnp.int32, (_T, 32), 1)
    h = lane // _NP
    lim = float(_SIDE - 1)

    x0c = jnp.clip(x0, 0.0, lim)
    y0c = jnp.clip(y0, 0.0, lim)
    wx0e = (jnp.where(x0c == x0, 1.0 - fx, 0.0)
            + jnp.where(x0c == x0 + 1.0, fx, 0.0))
    wx1e = jnp.where((x0c == x0) & (x0c + 1.0 <= lim), fx, 0.0)
    wy0e = (jnp.where(y0c == y0, 1.0 - fy, 0.0)
            + jnp.where(y0c == y0 + 1.0, fy, 0.0))
    wy1e = jnp.where((y0c == y0) & (y0c + 1.0 <= lim), fy, 0.0)

    xi = x0c.astype(jnp.int32)
    yi = y0c.astype(jnp.int32)
    idx_ref[...] = b * (_Q * _NH) + (yi * _SIDE + xi) * _NH + h
    c00 = wx0e * wy0e * attn
    c10 = wx1e * wy0e * attn
    c01 = wx0e * wy1e * attn
    c11 = wx1e * wy1e * attn
    cw_ref[...] = jnp.concatenate([c00, c10, c01, c11], axis=1)


def _prep(q, r, wcat, bcat, smat):
    return pl.pallas_call(
        _prep_body,
        grid=(_GRID,),
        in_specs=[
            pl.BlockSpec((_T, _D), lambda i: (i, 0)),
            pl.BlockSpec((_T, 2), lambda i: (i, 0)),
            pl.BlockSpec((_D, 96), lambda i: (0, 0)),
            pl.BlockSpec((1, 96), lambda i: (0, 0)),
            pl.BlockSpec((32, 32), lambda i: (0, 0)),
        ],
        out_specs=[
            pl.BlockSpec((_T, 32), lambda i: (i, 0)),
            pl.BlockSpec((_T, 128), lambda i: (i, 0)),
        ],
        out_shape=[
            jax.ShapeDtypeStruct((_ROWS, 32), jnp.int32),
            jax.ShapeDtypeStruct((_ROWS, 128), jnp.float32),
        ],
    )(q, r, wcat, bcat, smat)


def _bcast_lane(v, j):
    idx = jnp.full((16, 1), j, dtype=jnp.int32)
    dn = lax.GatherDimensionNumbers(
        offset_dims=(), collapsed_slice_dims=(0,), start_index_map=(0,))
    return lax.gather(v, idx, dn, (1,),
                      mode=lax.GatherScatterMode.PROMISE_IN_BOUNDS)


def _sc_body(table_hbm, idx_hbm, cw_hbm, out_hbm,
             idx_v, cw_v, rows_v, out_v, gsem):
    wid = lax.axis_index("s") * 2 + lax.axis_index("c")
    nk = (_NCHUNK - wid + _NW - 1) // _NW

    def chunk(k, carry):
        row0 = (wid + k * _NW) * _CH
        pltpu.sync_copy(idx_hbm.at[pl.ds(row0, _CH)], idx_v)
        pltpu.sync_copy(cw_hbm.at[pl.ds(row0, _CH)], cw_v)
        copies = [pltpu.make_async_copy(table_hbm.at[idx_v.at[j]],
                                        rows_v.at[j], gsem)
                  for j in range(_CH)]
        for c in copies:
            c.start()
        for c in copies:
            c.wait()

        def qbody(q, carry2):
            wv = [cw_v[q, pl.ds(kk * 16, 16)] for kk in range(8)]
            for h in range(_NH):
                acc0 = jnp.zeros((16,), jnp.float32)
                acc1 = jnp.zeros((16,), jnp.float32)
                for p in range(_NP):
                    ln = h * _NP + p
                    for c4 in range(4):
                        wl = c4 * 32 + h * _NP + p
                        w = _bcast_lane(wv[wl // 16], wl % 16)
                        acc0 = acc0 + w * rows_v[q, ln, pl.ds(c4 * 32, 16)]
                        acc1 = acc1 + w * rows_v[q, ln, pl.ds(c4 * 32 + 16, 16)]
                out_v[q * _NH + h, pl.ds(0, 16)] = acc0
                out_v[q * _NH + h, pl.ds(16, 16)] = acc1
            return carry2

        lax.fori_loop(0, _CH, qbody, 0)
        pltpu.sync_copy(out_v, out_hbm.at[pl.ds(row0 * _NH, _CH * _NH)])
        return carry

    lax.fori_loop(0, nk, chunk, 0)


@functools.cache
def _sc_gather_fn():
    return functools.partial(
        pl.kernel,
        mesh=plsc.VectorSubcoreMesh(core_axis_name="c", subcore_axis_name="s"),
        out_type=jax.ShapeDtypeStruct((_ROWS * _NH, _HD), jnp.float32),
        scratch_types=[
            pltpu.VMEM((_CH, 32), jnp.int32),
            pltpu.VMEM((_CH, 128), jnp.float32),
            pltpu.VMEM((_CH, 32, 4 * _HD), jnp.float32),
            pltpu.VMEM((_CH * _NH, _HD), jnp.float32),
            pltpu.SemaphoreType.DMA,
        ],
    )(_sc_body)


def kernel(query, value, ref_2d, spatial_shapes, level_start_index,
           W_off, b_off, W_attn, b_attn, W_val, b_val, W_out, b_out):
    q2 = query.reshape(_ROWS, _D)
    v2 = value.reshape(_ROWS, _D)
    r2 = ref_2d.reshape(_ROWS, 2)

    wo = W_off.reshape(_NH, _NP, 2, _D)
    wcat = jnp.concatenate(
        [wo[:, :, 0, :].reshape(32, _D),
         wo[:, :, 1, :].reshape(32, _D),
         W_attn], axis=0).T
    bo = b_off.reshape(_NH, _NP, 2)
    bcat = jnp.concatenate(
        [bo[:, :, 0].reshape(32), bo[:, :, 1].reshape(32), b_attn],
        axis=0).reshape(1, 96)
    smat = jnp.asarray(
        np.kron(np.eye(_NH, dtype=np.float32),
                np.ones((_NP, _NP), dtype=np.float32)))

    v_pad = jnp.concatenate(
        [v2, jnp.zeros((_TPAD, _D), jnp.float32)], axis=0)
    table = _vq(v_pad, W_val.T, b_val.reshape(1, _D))
    idx, cw = _prep(q2, r2, wcat, bcat, smat)
    sampled = _sc_gather_fn()(table.reshape(_ROWS * _NH, 4 * _HD), idx, cw)
    out = _mm_bias_res(sampled.reshape(_ROWS, _D), W_out.T,
                       b_out.reshape(1, _D), q2)
    return out.reshape(_BS, _Q, _D)


# speedup vs baseline: 2.0759x; 2.0759x over previous
import functools

import jax
import jax.numpy as jnp
import numpy as np
from jax import lax
from jax.experimental import pallas as pl
from jax.experimental.pallas import tpu as pltpu
from jax.experimental.pallas import tpu_sc as plsc

_BS = 2
_Q = 10000
_D = 256
_NH = 8
_NP = 4
_HD = 32
_SIDE = 100
_ROWS = _BS * _Q
_T = 400
_GRID = _ROWS // _T
_NW = 32
_CH = 8
_NCHUNK = _ROWS // _CH


def _mm_bias_body(x_ref, w_ref, b_ref, o_ref):
    o_ref[...] = (jnp.dot(x_ref[...], w_ref[...],
                          preferred_element_type=jnp.float32) + b_ref[...])


def _mm_bias(x, w, b):
    return pl.pallas_call(
        _mm_bias_body,
        grid=(_GRID,),
        in_specs=[
            pl.BlockSpec((_T, _D), lambda i: (i, 0)),
            pl.BlockSpec((_D, _D), lambda i: (0, 0)),
            pl.BlockSpec((1, _D), lambda i: (0, 0)),
        ],
        out_specs=pl.BlockSpec((_T, _D), lambda i: (i, 0)),
        out_shape=jax.ShapeDtypeStruct((_ROWS, _D), jnp.float32),
    )(x, w, b)


def _mm_bias_res_body(x_ref, w_ref, b_ref, id_ref, o_ref):
    o_ref[...] = (jnp.dot(x_ref[...], w_ref[...],
                          preferred_element_type=jnp.float32)
                  + b_ref[...] + id_ref[...])


def _mm_bias_res(x, w, b, ident):
    return pl.pallas_call(
        _mm_bias_res_body,
        grid=(_GRID,),
        in_specs=[
            pl.BlockSpec((_T, _D), lambda i: (i, 0)),
            pl.BlockSpec((_D, _D), lambda i: (0, 0)),
            pl.BlockSpec((1, _D), lambda i: (0, 0)),
            pl.BlockSpec((_T, _D), lambda i: (i, 0)),
        ],
        out_specs=pl.BlockSpec((_T, _D), lambda i: (i, 0)),
        out_shape=jax.ShapeDtypeStruct((_ROWS, _D), jnp.float32),
    )(x, w, b, ident)


def _prep_body(q_ref, r_ref, wcat_ref, bcat_ref, s_ref, idx_ref, cw_ref):
    t = pl.program_id(0)
    b = t // (_GRID // _BS)
    s = (jnp.dot(q_ref[...], wcat_ref[...],
                 preferred_element_type=jnp.float32) + bcat_ref[...])
    ox = s[:, 0:32]
    oy = s[:, 32:64]
    lg = s[:, 64:96]
    e = jnp.exp(lg)
    denom = jnp.dot(e, s_ref[...], preferred_element_type=jnp.float32)
    attn = e / denom
    rx = r_ref[:, 0:1]
    ry = r_ref[:, 1:2]
    gx = rx * float(_SIDE) - 0.5 + ox
    gy = ry * float(_SIDE) - 0.5 + oy
    x0 = jnp.floor(gx)
    fx = gx - x0
    y0 = jnp.floor(gy)
    fy = gy - y0
    lane = lax.broadcasted_iota(jnp.int32, (_T, 32), 1)
    h = lane // _NP
    base = b * (_Q * _NH) + h

    def corner(xf, yf, w):
        lim = float(_SIDE - 1)
        v = (xf >= 0.0) & (xf <= lim) & (yf >= 0.0) & (yf <= lim)
        xi = jnp.clip(xf, 0.0, lim).astype(jnp.int32)
        yi = jnp.clip(yf, 0.0, lim).astype(jnp.int32)
        idx = base + (yi * _SIDE + xi) * _NH
        cw = jnp.where(v, w * attn, 0.0)
        return idx, cw

    i00, c00 = corner(x0, y0, (1.0 - fx) * (1.0 - fy))
    i10, c10 = corner(x0 + 1.0, y0, fx * (1.0 - fy))
    i01, c01 = corner(x0, y0 + 1.0, (1.0 - fx) * fy)
    i11, c11 = corner(x0 + 1.0, y0 + 1.0, fx * fy)
    idx_ref[...] = jnp.concatenate([i00, i10, i01, i11], axis=1)
    cw_ref[...] = jnp.concatenate([c00, c10, c01, c11], axis=1)


def _prep(q, r, wcat, bcat, smat):
    return pl.pallas_call(
        _prep_body,
        grid=(_GRID,),
        in_specs=[
            pl.BlockSpec((_T, _D), lambda i: (i, 0)),
            pl.BlockSpec((_T, 2), lambda i: (i, 0)),
            pl.BlockSpec((_D, 96), lambda i: (0, 0)),
            pl.BlockSpec((1, 96), lambda i: (0, 0)),
            pl.BlockSpec((32, 32), lambda i: (0, 0)),
        ],
        out_specs=[
            pl.BlockSpec((_T, 128), lambda i: (i, 0)),
            pl.BlockSpec((_T, 128), lambda i: (i, 0)),
        ],
        out_shape=[
            jax.ShapeDtypeStruct((_ROWS, 128), jnp.int32),
            jax.ShapeDtypeStruct((_ROWS, 128), jnp.float32),
        ],
    )(q, r, wcat, bcat, smat)


def _bcast_lane(v, j):
    idx = jnp.full((16, 1), j, dtype=jnp.int32)
    dn = lax.GatherDimensionNumbers(
        offset_dims=(), collapsed_slice_dims=(0,), start_index_map=(0,))
    return lax.gather(v, idx, dn, (1,),
                      mode=lax.GatherScatterMode.PROMISE_IN_BOUNDS)


def _sc_body(table_hbm, idx_hbm, cw_hbm, out_hbm,
             idx_v, cw_v, rows_v, out_v, gsem, isem, osem):
    wid = lax.axis_index("s") * 2 + lax.axis_index("c")
    nk = (_NCHUNK - wid + _NW - 1) // _NW

    def row0_of(k):
        return (wid + k * _NW) * _CH

    def copy_in_descs(k):
        buf = k % 2
        r0 = row0_of(k)
        return (
            pltpu.make_async_copy(idx_hbm.at[pl.ds(r0, _CH)],
                                  idx_v.at[buf], isem),
            pltpu.make_async_copy(cw_hbm.at[pl.ds(r0, _CH)],
                                  cw_v.at[buf], isem),
        )

    def gather_descs(k):
        buf = k % 2
        return [pltpu.make_async_copy(table_hbm.at[idx_v.at[buf, j]],
                                      rows_v.at[buf, j], gsem)
                for j in range(_CH)]

    def out_desc(k):
        buf = k % 2
        return pltpu.make_async_copy(
            out_v.at[buf], out_hbm.at[pl.ds(row0_of(k) * _NH, _CH * _NH)],
            osem)

    def start_all(descs):
        for d in descs:
            d.start()

    def wait_all(descs):
        for d in descs:
            d.wait()

    start_all(copy_in_descs(0))
    wait_all(copy_in_descs(0))
    start_all(gather_descs(0))
    start_all(copy_in_descs(1))

    def step(k, carry):
        buf = k % 2

        @pl.when(k + 1 < nk)
        def _():
            wait_all(copy_in_descs(k + 1))
            start_all(gather_descs(k + 1))

        @pl.when(k + 2 < nk)
        def _():
            start_all(copy_in_descs(k + 2))

        wait_all(gather_descs(k))

        @pl.when(k >= 2)
        def _():
            out_desc(k - 2).wait()

        def qbody(q, carry2):
            wv = [cw_v[buf, q, pl.ds(kk * 16, 16)] for kk in range(8)]
            for h in range(_NH):
                acc0 = jnp.zeros((16,), jnp.float32)
                acc1 = jnp.zeros((16,), jnp.float32)
                for c4 in range(4):
                    for p in range(_NP):
                        ln = c4 * 32 + h * _NP + p
                        w = _bcast_lane(wv[ln // 16], ln % 16)
                        acc0 = acc0 + w * rows_v[buf, q, ln, pl.ds(0, 16)]
                        acc1 = acc1 + w * rows_v[buf, q, ln, pl.ds(16, 16)]
                out_v[buf, q * _NH + h, pl.ds(0, 16)] = acc0
                out_v[buf, q * _NH + h, pl.ds(16, 16)] = acc1
            return carry2

        lax.fori_loop(0, _CH, qbody, 0)
        out_desc(k).start()
        return carry

    lax.fori_loop(0, nk, step, 0)
    out_desc(nk - 2).wait()
    out_desc(nk - 1).wait()


@functools.cache
def _sc_gather_fn():
    return functools.partial(
        pl.kernel,
        mesh=plsc.VectorSubcoreMesh(core_axis_name="c", subcore_axis_name="s"),
        compiler_params=pltpu.CompilerParams(use_tc_tiling_on_sc=False),
        out_type=jax.ShapeDtypeStruct((_ROWS * _NH, _HD), jnp.float32),
        scratch_types=[
            pltpu.VMEM((2, _CH, 128), jnp.int32),
            pltpu.VMEM((2, _CH, 128), jnp.float32),
            pltpu.VMEM((2, _CH, 128, _HD), jnp.float32),
            pltpu.VMEM((2, _CH * _NH, _HD), jnp.float32),
            pltpu.SemaphoreType.DMA,
            pltpu.SemaphoreType.DMA,
            pltpu.SemaphoreType.DMA,
        ],
    )(_sc_body)


def kernel(query, value, ref_2d, spatial_shapes, level_start_index,
           W_off, b_off, W_attn, b_attn, W_val, b_val, W_out, b_out):
    q2 = query.reshape(_ROWS, _D)
    v2 = value.reshape(_ROWS, _D)
    r2 = ref_2d.reshape(_ROWS, 2)

    wo = W_off.reshape(_NH, _NP, 2, _D)
    wcat = jnp.concatenate(
        [wo[:, :, 0, :].reshape(32, _D),
         wo[:, :, 1, :].reshape(32, _D),
         W_attn], axis=0).T
    bo = b_off.reshape(_NH, _NP, 2)
    bcat = jnp.concatenate(
        [bo[:, :, 0].reshape(32), bo[:, :, 1].reshape(32), b_attn],
        axis=0).reshape(1, 96)
    smat = jnp.asarray(
        np.kron(np.eye(_NH, dtype=np.float32),
                np.ones((_NP, _NP), dtype=np.float32)))

    table = _mm_bias(v2, W_val.T, b_val.reshape(1, _D))
    idx, cw = _prep(q2, r2, wcat, bcat, smat)
    sampled = _sc_gather_fn()(table.reshape(_ROWS * _NH, _HD), idx, cw)
    out = _mm_bias_res(sampled.reshape(_ROWS, _D), W_out.T,
                       b_out.reshape(1, _D), q2)
    return out.reshape(_BS, _Q, _D)
